# baseline (speedup 1.0000x reference)
import jax
import jax.numpy as jnp
from jax import lax
from jax.experimental import pallas as pl
from jax.experimental.pallas import tpu as pltpu
from jax.experimental.pallas import tpu_sc as plsc

N = 10000
E = 320000
IN_F = 128
H_F = 128
OUT_F = 64

NC = 2
NS = 16
NW = NC * NS
CHUNK = 128
NCH = 80
E_PER_TILE = CHUNK * NCH
E_PAD = NW * E_PER_TILE
N_PAD = 10112
RPT = N_PAD // NS

_MESH = plsc.VectorSubcoreMesh(core_axis_name="c", subcore_axis_name="s")
_SC_PARAMS = pltpu.CompilerParams(use_tc_tiling_on_sc=False)


def _deg_call(dstw, ones_d, zeros_d):
    def body(dstw_hbm, ones_hbm, zeros_hbm, out_hbm, dst_v, ones_v, acc_sh, sem):
        c = lax.axis_index("c")
        s = lax.axis_index("s")
        wid = c * NS + s
        pltpu.sync_copy(zeros_hbm.at[pl.ds(s * RPT, RPT)],
                        acc_sh.at[pl.ds(s * RPT, RPT)])
        pltpu.sync_copy(dstw_hbm.at[wid], dst_v)
        pltpu.sync_copy(ones_hbm, ones_v)
        plsc.subcore_barrier()

        @pl.loop(0, NCH)
        def _(j):
            pltpu.async_copy(ones_v, acc_sh.at[dst_v.at[j]], sem, add=True).wait()

        plsc.subcore_barrier()
        pltpu.sync_copy(acc_sh.at[pl.ds(s * RPT, RPT)],
                        out_hbm.at[c].at[pl.ds(s * RPT, RPT)])

    fn = pl.kernel(
        body,
        out_type=jax.ShapeDtypeStruct((NC, N_PAD, 16), jnp.float32),
        mesh=_MESH,
        scratch_types=[
            pltpu.VMEM((NCH, CHUNK), jnp.int32),
            pltpu.VMEM((CHUNK, 16), jnp.float32),
            pltpu.VMEM_SHARED((N_PAD, 16), jnp.float32),
            pltpu.SemaphoreType.DMA,
        ],
        compiler_params=_SC_PARAMS,
    )
    return fn(dstw, ones_d, zeros_d)


def _edge_sum_call(y, srcw, dstw, zeros, d):
    def body(y_hbm, srcw_hbm, dstw_hbm, zeros_hbm, out_hbm,
             src_v, dst_v, rows_a, rows_b, acc_sh,
             sem_ia, sem_ib, sem_ga, sem_gb, sem_s):
        c = lax.axis_index("c")
        s = lax.axis_index("s")
        wid = (1 - c) * NS + s
        pltpu.sync_copy(zeros_hbm.at[pl.ds(s * RPT, RPT)],
                        acc_sh.at[pl.ds(s * RPT, RPT)])
        plsc.subcore_barrier()

        rows = (rows_a, rows_b)
        isems = (sem_ia, sem_ib)
        gsems = (sem_ga, sem_gb)

        def i_start(j, slot):
            pltpu.make_async_copy(srcw_hbm.at[wid].at[j], src_v.at[slot],
                                  isems[slot]).start()
            pltpu.make_async_copy(dstw_hbm.at[wid].at[j], dst_v.at[slot],
                                  isems[slot]).start()

        def i_wait(j, slot):
            pltpu.make_async_copy(srcw_hbm.at[wid].at[j], src_v.at[slot],
                                  isems[slot]).wait()
            pltpu.make_async_copy(dstw_hbm.at[wid].at[j], dst_v.at[slot],
                                  isems[slot]).wait()

        def g_start(slot):
            pltpu.make_async_copy(y_hbm.at[src_v.at[slot]], rows[slot],
                                  gsems[slot]).start()

        def g_wait(slot):
            pltpu.make_async_copy(y_hbm.at[src_v.at[slot]], rows[slot],
                                  gsems[slot]).wait()

        i_start(0, 0)
        i_start(1, 1)

        @pl.loop(0, NCH // 2)
        def _(jh):
            j = jh * 2
            i_wait(j, 0)
            g_start(0)
            i_wait(j + 1, 1)
            g_start(1)
            g_wait(0)
            sc_a = pltpu.async_copy(rows_a, acc_sh.at[dst_v.at[0]], sem_s,
                                    add=True)
            g_wait(1)
            sc_a.wait()

            @pl.when(j + 2 < NCH)
            def _():
                i_start(j + 2, 0)

            sc_b = pltpu.async_copy(rows_b, acc_sh.at[dst_v.at[1]], sem_s,
                                    add=True)
            sc_b.wait()

            @pl.when(j + 3 < NCH)
            def _():
                i_start(j + 3, 1)

        plsc.subcore_barrier()
        pltpu.sync_copy(acc_sh.at[pl.ds(s * RPT, RPT)],
                        out_hbm.at[c].at[pl.ds(s * RPT, RPT)])

    fn = pl.kernel(
        body,
        out_type=jax.ShapeDtypeStruct((NC, N_PAD, d), jnp.float32),
        mesh=_MESH,
        scratch_types=[
            pltpu.VMEM((2, CHUNK), jnp.int32),
            pltpu.VMEM((2, CHUNK), jnp.int32),
            pltpu.VMEM((CHUNK, d), jnp.float32),
            pltpu.VMEM((CHUNK, d), jnp.float32),
            pltpu.VMEM_SHARED((N_PAD, d), jnp.float32),
            pltpu.SemaphoreType.DMA,
            pltpu.SemaphoreType.DMA,
            pltpu.SemaphoreType.DMA,
            pltpu.SemaphoreType.DMA,
            pltpu.SemaphoreType.DMA,
        ],
        compiler_params=_SC_PARAMS,
    )
    return fn(y, srcw, dstw, zeros)


BLK = 2000


def _mm1_call(x, W1):
    def body(x_ref, w_ref, o_ref):
        o_ref[...] = jnp.dot(x_ref[...], w_ref[...],
                             preferred_element_type=jnp.float32)

    return pl.pallas_call(
        body,
        grid=(N // BLK,),
        in_specs=[pl.BlockSpec((BLK, IN_F), lambda i: (i, 0)),
                  pl.BlockSpec((IN_F, H_F), lambda i: (0, 0))],
        out_specs=pl.BlockSpec((BLK, H_F), lambda i: (i, 0)),
        out_shape=jax.ShapeDtypeStruct((N, H_F), jnp.float32),
    )(x, W1)


def _ew1_call(degp, xw1):
    def body(degp_ref, xw_ref, y_ref, dinv_ref):
        deg = degp_ref[0, :, 0:1] + degp_ref[1, :, 0:1]
        dinv = lax.rsqrt(deg + 1.0)
        y_ref[...] = xw_ref[...] * dinv
        dinv_ref[...] = dinv

    return pl.pallas_call(
        body,
        grid=(N // BLK,),
        in_specs=[pl.BlockSpec((NC, BLK, 16), lambda i: (0, i, 0)),
                  pl.BlockSpec((BLK, H_F), lambda i: (i, 0))],
        out_specs=[pl.BlockSpec((BLK, H_F), lambda i: (i, 0)),
                   pl.BlockSpec((BLK, 1), lambda i: (i, 0))],
        out_shape=[jax.ShapeDtypeStruct((N, H_F), jnp.float32),
                   jax.ShapeDtypeStruct((N, 1), jnp.float32)],
    )(degp, xw1)


def _fused2_call(s1p, y1, dinv, b1, W2):
    def body(s_ref, y_ref, dinv_ref, b_ref, w_ref, o_ref):
        sacc = s_ref[0] + s_ref[1]
        h = jnp.maximum(dinv_ref[...] * (sacc + y_ref[...]) + b_ref[...], 0.0)
        o_ref[...] = jnp.dot(h, w_ref[...],
                             preferred_element_type=jnp.float32) * dinv_ref[...]

    return pl.pallas_call(
        body,
        grid=(N // BLK,),
        in_specs=[pl.BlockSpec((NC, BLK, H_F), lambda i: (0, i, 0)),
                  pl.BlockSpec((BLK, H_F), lambda i: (i, 0)),
                  pl.BlockSpec((BLK, 1), lambda i: (i, 0)),
                  pl.BlockSpec((1, H_F), lambda i: (0, 0)),
                  pl.BlockSpec((H_F, OUT_F), lambda i: (0, 0))],
        out_specs=pl.BlockSpec((BLK, OUT_F), lambda i: (i, 0)),
        out_shape=jax.ShapeDtypeStruct((N, OUT_F), jnp.float32),
    )(s1p, y1, dinv, b1, W2)


def _ew3_call(s2p, y2, dinv, b2):
    def body(s_ref, y_ref, dinv_ref, b_ref, o_ref):
        sacc = s_ref[0] + s_ref[1]
        o_ref[...] = dinv_ref[...] * (sacc + y_ref[...]) + b_ref[...]

    return pl.pallas_call(
        body,
        grid=(N // BLK,),
        in_specs=[pl.BlockSpec((NC, BLK, OUT_F), lambda i: (0, i, 0)),
                  pl.BlockSpec((BLK, OUT_F), lambda i: (i, 0)),
                  pl.BlockSpec((BLK, 1), lambda i: (i, 0)),
                  pl.BlockSpec((1, OUT_F), lambda i: (0, 0))],
        out_specs=pl.BlockSpec((BLK, OUT_F), lambda i: (i, 0)),
        out_shape=jax.ShapeDtypeStruct((N, OUT_F), jnp.float32),
    )(s2p, y2, dinv, b2)


def kernel(x, edge_index, W1, b1, W2, b2):
    src = edge_index[0].astype(jnp.int32)
    dst = edge_index[1].astype(jnp.int32)
    pad = E_PAD - E
    srcw = jnp.concatenate([src, jnp.zeros((pad,), jnp.int32)]
                           ).reshape(NW, NCH, CHUNK)
    trash = N + (jnp.arange(pad, dtype=jnp.int32) % (N_PAD - N))
    dstw = jnp.concatenate([dst, trash]).reshape(NW, NCH, CHUNK)

    ones_d = jnp.ones((CHUNK, 16), jnp.float32)
    zeros_d = jnp.zeros((N_PAD, 16), jnp.float32)
    zeros_h = jnp.zeros((N_PAD, H_F), jnp.float32)
    zeros_o = jnp.zeros((N_PAD, OUT_F), jnp.float32)

    degp = _deg_call(dstw, ones_d, zeros_d)
    xw1 = _mm1_call(x, W1)
    y1, dinv = _ew1_call(degp, xw1)
    s1p = _edge_sum_call(y1, srcw, dstw, zeros_h, H_F)
    y2 = _fused2_call(s1p, y1, dinv, b1.reshape(1, H_F), W2)
    s2p = _edge_sum_call(y2, srcw, dstw, zeros_o, OUT_F)
    out = _ew3_call(s2p, y2, dinv, b2.reshape(1, OUT_F))
    return out


# speedup vs baseline: 2.6778x; 2.6778x over previous
import jax
import jax.numpy as jnp
from jax import lax
from jax.experimental import pallas as pl
from jax.experimental.pallas import tpu as pltpu
from jax.experimental.pallas import tpu_sc as plsc

N = 10000
E = 320000
IN_F = 128
H_F = 128
OUT_F = 64

NC = 2
NS = 16
NW = NC * NS
CHUNK = 128
NCH = 80
E_PER_TILE = CHUNK * NCH
E_PAD = NW * E_PER_TILE
N_PAD = 10112
RPT = N_PAD // NS

_MESH = plsc.VectorSubcoreMesh(core_axis_name="c", subcore_axis_name="s")
_SC_PARAMS = pltpu.CompilerParams(use_tc_tiling_on_sc=False)


def _deg_call(dstw, ones_d, zeros_d):
    def body(dstw_hbm, ones_hbm, zeros_hbm, out_hbm, dst_v, ones_v, acc_sh, sem):
        c = lax.axis_index("c")
        s = lax.axis_index("s")
        wid = c * NS + s
        pltpu.sync_copy(zeros_hbm.at[pl.ds(s * RPT, RPT)],
                        acc_sh.at[pl.ds(s * RPT, RPT)])
        pltpu.sync_copy(dstw_hbm.at[wid], dst_v)
        pltpu.sync_copy(ones_hbm, ones_v)
        plsc.subcore_barrier()

        @pl.loop(0, NCH)
        def _(j):
            pltpu.async_copy(ones_v, acc_sh.at[dst_v.at[j]], sem, add=True).wait()

        plsc.subcore_barrier()
        pltpu.sync_copy(acc_sh.at[pl.ds(s * RPT, RPT)],
                        out_hbm.at[c].at[pl.ds(s * RPT, RPT)])

    fn = pl.kernel(
        body,
        out_type=jax.ShapeDtypeStruct((NC, N_PAD, 16), jnp.float32),
        mesh=_MESH,
        scratch_types=[
            pltpu.VMEM((NCH, CHUNK), jnp.int32),
            pltpu.VMEM((CHUNK, 16), jnp.float32),
            pltpu.VMEM_SHARED((N_PAD, 16), jnp.float32),
            pltpu.SemaphoreType.DMA,
        ],
        compiler_params=_SC_PARAMS,
    )
    return fn(dstw, ones_d, zeros_d)


def _edge_sum_call(y, srcw, dstw, zeros, d):
    def body(y_hbm, srcw_hbm, dstw_hbm, zeros_hbm, out_hbm,
             src_v, dst_v, rows_a, rows_b, acc_sh,
             sem_ia, sem_ib, sem_ga, sem_gb, sem_s):
        c = lax.axis_index("c")
        s = lax.axis_index("s")
        wid = c * NS + s
        pltpu.sync_copy(zeros_hbm.at[pl.ds(s * RPT, RPT)],
                        acc_sh.at[pl.ds(s * RPT, RPT)])
        plsc.subcore_barrier()

        rows = (rows_a, rows_b)
        isems = (sem_ia, sem_ib)
        gsems = (sem_ga, sem_gb)

        def i_start(j, slot):
            pltpu.make_async_copy(srcw_hbm.at[wid].at[j], src_v.at[slot],
                                  isems[slot]).start()
            pltpu.make_async_copy(dstw_hbm.at[wid].at[j], dst_v.at[slot],
                                  isems[slot]).start()

        def i_wait(j, slot):
            pltpu.make_async_copy(srcw_hbm.at[wid].at[j], src_v.at[slot],
                                  isems[slot]).wait()
            pltpu.make_async_copy(dstw_hbm.at[wid].at[j], dst_v.at[slot],
                                  isems[slot]).wait()

        def g_start(slot):
            pltpu.make_async_copy(y_hbm.at[src_v.at[slot]], rows[slot],
                                  gsems[slot]).start()

        def g_wait(slot):
            pltpu.make_async_copy(y_hbm.at[src_v.at[slot]], rows[slot],
                                  gsems[slot]).wait()

        i_start(0, 0)
        i_start(1, 1)

        @pl.loop(0, NCH // 2)
        def _(jh):
            j = jh * 2
            i_wait(j, 0)
            g_start(0)
            i_wait(j + 1, 1)
            g_start(1)
            g_wait(0)
            sc_a = pltpu.async_copy(rows_a, acc_sh.at[dst_v.at[0]], sem_s,
                                    add=True)
            g_wait(1)
            sc_a.wait()

            @pl.when(j + 2 < NCH)
            def _():
                i_start(j + 2, 0)

            sc_b = pltpu.async_copy(rows_b, acc_sh.at[dst_v.at[1]], sem_s,
                                    add=True)
            sc_b.wait()

            @pl.when(j + 3 < NCH)
            def _():
                i_start(j + 3, 1)

        plsc.subcore_barrier()
        pltpu.sync_copy(acc_sh.at[pl.ds(s * RPT, RPT)],
                        out_hbm.at[c].at[pl.ds(s * RPT, RPT)])

    fn = pl.kernel(
        body,
        out_type=jax.ShapeDtypeStruct((NC, N_PAD, d), jnp.float32),
        mesh=_MESH,
        scratch_types=[
            pltpu.VMEM((2, CHUNK), jnp.int32),
            pltpu.VMEM((2, CHUNK), jnp.int32),
            pltpu.VMEM((CHUNK, d), jnp.float32),
            pltpu.VMEM((CHUNK, d), jnp.float32),
            pltpu.VMEM_SHARED((N_PAD, d), jnp.float32),
            pltpu.SemaphoreType.DMA,
            pltpu.SemaphoreType.DMA,
            pltpu.SemaphoreType.DMA,
            pltpu.SemaphoreType.DMA,
            pltpu.SemaphoreType.DMA,
        ],
        compiler_params=_SC_PARAMS,
    )
    return fn(y, srcw, dstw, zeros)


BLK = 2000


def _mm1_call(x, W1):
    def body(x_ref, w_ref, o_ref):
        o_ref[...] = jnp.dot(x_ref[...], w_ref[...],
                             preferred_element_type=jnp.float32)

    return pl.pallas_call(
        body,
        grid=(N // BLK,),
        in_specs=[pl.BlockSpec((BLK, IN_F), lambda i: (i, 0)),
                  pl.BlockSpec((IN_F, H_F), lambda i: (0, 0))],
        out_specs=pl.BlockSpec((BLK, H_F), lambda i: (i, 0)),
        out_shape=jax.ShapeDtypeStruct((N, H_F), jnp.float32),
    )(x, W1)


def _ew1_call(degp, xw1):
    def body(degp_ref, xw_ref, y_ref, dinv_ref):
        deg = degp_ref[0, :, 0:1] + degp_ref[1, :, 0:1]
        dinv = lax.rsqrt(deg + 1.0)
        y_ref[...] = xw_ref[...] * dinv
        dinv_ref[...] = dinv

    return pl.pallas_call(
        body,
        grid=(N // BLK,),
        in_specs=[pl.BlockSpec((NC, BLK, 16), lambda i: (0, i, 0)),
                  pl.BlockSpec((BLK, H_F), lambda i: (i, 0))],
        out_specs=[pl.BlockSpec((BLK, H_F), lambda i: (i, 0)),
                   pl.BlockSpec((BLK, 1), lambda i: (i, 0))],
        out_shape=[jax.ShapeDtypeStruct((N, H_F), jnp.float32),
                   jax.ShapeDtypeStruct((N, 1), jnp.float32)],
    )(degp, xw1)


def _fused2_call(s1p, y1, dinv, b1, W2):
    def body(s_ref, y_ref, dinv_ref, b_ref, w_ref, o_ref):
        sacc = s_ref[0] + s_ref[1]
        h = jnp.maximum(dinv_ref[...] * (sacc + y_ref[...]) + b_ref[...], 0.0)
        o_ref[...] = jnp.dot(h, w_ref[...],
                             preferred_element_type=jnp.float32) * dinv_ref[...]

    return pl.pallas_call(
        body,
        grid=(N // BLK,),
        in_specs=[pl.BlockSpec((NC, BLK, H_F), lambda i: (0, i, 0)),
                  pl.BlockSpec((BLK, H_F), lambda i: (i, 0)),
                  pl.BlockSpec((BLK, 1), lambda i: (i, 0)),
                  pl.BlockSpec((1, H_F), lambda i: (0, 0)),
                  pl.BlockSpec((H_F, OUT_F), lambda i: (0, 0))],
        out_specs=pl.BlockSpec((BLK, OUT_F), lambda i: (i, 0)),
        out_shape=jax.ShapeDtypeStruct((N, OUT_F), jnp.float32),
    )(s1p, y1, dinv, b1, W2)


def _ew3_call(s2p, y2, dinv, b2):
    def body(s_ref, y_ref, dinv_ref, b_ref, o_ref):
        sacc = s_ref[0] + s_ref[1]
        o_ref[...] = dinv_ref[...] * (sacc + y_ref[...]) + b_ref[...]

    return pl.pallas_call(
        body,
        grid=(N // BLK,),
        in_specs=[pl.BlockSpec((NC, BLK, OUT_F), lambda i: (0, i, 0)),
                  pl.BlockSpec((BLK, OUT_F), lambda i: (i, 0)),
                  pl.BlockSpec((BLK, 1), lambda i: (i, 0)),
                  pl.BlockSpec((1, OUT_F), lambda i: (0, 0))],
        out_specs=pl.BlockSpec((BLK, OUT_F), lambda i: (i, 0)),
        out_shape=jax.ShapeDtypeStruct((N, OUT_F), jnp.float32),
    )(s2p, y2, dinv, b2)


def kernel(x, edge_index, W1, b1, W2, b2):
    src = edge_index[0].astype(jnp.int32)
    dst = edge_index[1].astype(jnp.int32)
    pad = E_PAD - E
    ar = jnp.arange(pad, dtype=jnp.int32)
    srcw = jnp.concatenate([src, ar % N]).reshape(NW, NCH, CHUNK)
    dstw = jnp.concatenate([dst, N + ar % (N_PAD - N)]).reshape(NW, NCH, CHUNK)

    ones_d = jnp.ones((CHUNK, 16), jnp.float32)
    zeros_d = jnp.zeros((N_PAD, 16), jnp.float32)
    zeros_h = jnp.zeros((N_PAD, H_F), jnp.float32)
    zeros_o = jnp.zeros((N_PAD, OUT_F), jnp.float32)

    degp = _deg_call(dstw, ones_d, zeros_d)
    xw1 = _mm1_call(x, W1)
    y1, dinv = _ew1_call(degp, xw1)
    s1p = _edge_sum_call(y1, srcw, dstw, zeros_h, H_F)
    y2 = _fused2_call(s1p, y1, dinv, b1.reshape(1, H_F), W2)
    s2p = _edge_sum_call(y2, srcw, dstw, zeros_o, OUT_F)
    out = _ew3_call(s2p, y2, dinv, b2.reshape(1, OUT_F))
    return out


# speedup vs baseline: 2.7035x; 1.0096x over previous
import jax
import jax.numpy as jnp
from jax import lax
from jax.experimental import pallas as pl
from jax.experimental.pallas import tpu as pltpu
from jax.experimental.pallas import tpu_sc as plsc

N = 10000
E = 320000
IN_F = 128
H_F = 128
OUT_F = 64

NC = 2
NS = 16
NW = NC * NS
CHUNK = 128
NCHUNKS = E // CHUNK
NCH_MAX = (NCHUNKS + NW - 1) // NW
N_PAD = 10112
RPT = N_PAD // NS

_MESH = plsc.VectorSubcoreMesh(core_axis_name="c", subcore_axis_name="s")
_SC_PARAMS = pltpu.CompilerParams(use_tc_tiling_on_sc=False)


def _tile_range(wid):
    lo = wid * NCHUNKS // NW
    hi = (wid + 1) * NCHUNKS // NW
    return lo, hi - lo


def _deg_call(dst2, ones_d, zeros_d):
    def body(dst_hbm, ones_hbm, zeros_hbm, out_hbm, dst_v, ones_v, acc_sh, sem):
        c = lax.axis_index("c")
        s = lax.axis_index("s")
        wid = c * NS + s
        lo, cnt = _tile_range(wid)
        pltpu.sync_copy(zeros_hbm.at[pl.ds(s * RPT, RPT)],
                        acc_sh.at[pl.ds(s * RPT, RPT)])
        pltpu.sync_copy(dst_hbm.at[pl.ds(lo, NCH_MAX)], dst_v)
        pltpu.sync_copy(ones_hbm, ones_v)
        plsc.subcore_barrier()

        @pl.loop(0, NCH_MAX)
        def _(j):
            @pl.when(j < cnt)
            def _():
                pltpu.async_copy(ones_v, acc_sh.at[dst_v.at[j]], sem,
                                 add=True).wait()

        plsc.subcore_barrier()
        pltpu.sync_copy(acc_sh.at[pl.ds(s * RPT, RPT)],
                        out_hbm.at[c].at[pl.ds(s * RPT, RPT)])

    fn = pl.kernel(
        body,
        out_type=jax.ShapeDtypeStruct((NC, N_PAD, 16), jnp.float32),
        mesh=_MESH,
        scratch_types=[
            pltpu.VMEM((NCH_MAX, CHUNK), jnp.int32),
            pltpu.VMEM((CHUNK, 16), jnp.float32),
            pltpu.VMEM_SHARED((N_PAD, 16), jnp.float32),
            pltpu.SemaphoreType.DMA,
        ],
        compiler_params=_SC_PARAMS,
    )
    return fn(dst2, ones_d, zeros_d)


def _edge_sum_call(y, src2, dst2, zeros, d):
    def body(y_hbm, src_hbm, dst_hbm, zeros_hbm, out_hbm,
             src_v, dst_v, rows_a, rows_b,
             sem_ia, sem_ib, sem_ga, sem_gb, sem_s, acc_sh):
        c = lax.axis_index("c")
        s = lax.axis_index("s")
        wid = c * NS + s
        lo, cnt = _tile_range(wid)
        pltpu.sync_copy(zeros_hbm.at[pl.ds(s * RPT, RPT)],
                        acc_sh.at[pl.ds(s * RPT, RPT)])
        plsc.subcore_barrier()

        rows = (rows_a, rows_b)
        isems = (sem_ia, sem_ib)
        gsems = (sem_ga, sem_gb)

        def i_start(j, slot):
            pltpu.make_async_copy(src_hbm.at[j], src_v.at[slot],
                                  isems[slot]).start()
            pltpu.make_async_copy(dst_hbm.at[j], dst_v.at[slot],
                                  isems[slot]).start()

        def i_wait(j, slot):
            pltpu.make_async_copy(src_hbm.at[j], src_v.at[slot],
                                  isems[slot]).wait()
            pltpu.make_async_copy(dst_hbm.at[j], dst_v.at[slot],
                                  isems[slot]).wait()

        def g_start(slot):
            pltpu.make_async_copy(y_hbm.at[src_v.at[slot]], rows[slot],
                                  gsems[slot]).start()

        def g_wait(slot):
            pltpu.make_async_copy(y_hbm.at[src_v.at[slot]], rows[slot],
                                  gsems[slot]).wait()

        i_start(lo, 0)
        i_start(lo + 1, 1)

        @pl.loop(0, (NCH_MAX - 1) // 2)
        def _(jh):
            j = lo + 2 * jh
            i_wait(j, 0)
            g_start(0)
            i_wait(j + 1, 1)
            g_start(1)
            g_wait(0)
            sc_a = pltpu.async_copy(rows_a, acc_sh.at[dst_v.at[0]], sem_s,
                                    add=True)
            g_wait(1)
            sc_a.wait()

            @pl.when(2 * jh + 2 < cnt)
            def _():
                i_start(j + 2, 0)

            sc_b = pltpu.async_copy(rows_b, acc_sh.at[dst_v.at[1]], sem_s,
                                    add=True)
            sc_b.wait()

            @pl.when(2 * jh + 3 < cnt)
            def _():
                i_start(j + 3, 1)

        @pl.when(cnt > NCH_MAX - 1)
        def _():
            j = lo + NCH_MAX - 1
            i_wait(j, 0)
            g_start(0)
            g_wait(0)
            pltpu.async_copy(rows_a, acc_sh.at[dst_v.at[0]], sem_s,
                             add=True).wait()

        plsc.subcore_barrier()
        pltpu.sync_copy(acc_sh.at[pl.ds(s * RPT, RPT)],
                        out_hbm.at[c].at[pl.ds(s * RPT, RPT)])

    fn = pl.kernel(
        body,
        out_type=jax.ShapeDtypeStruct((NC, N_PAD, d), jnp.float32),
        mesh=_MESH,
        scratch_types=[
            pltpu.VMEM((2, CHUNK), jnp.int32),
            pltpu.VMEM((2, CHUNK), jnp.int32),
            pltpu.VMEM((CHUNK, d), jnp.float32),
            pltpu.VMEM((CHUNK, d), jnp.float32),
            pltpu.SemaphoreType.DMA,
            pltpu.SemaphoreType.DMA,
            pltpu.SemaphoreType.DMA,
            pltpu.SemaphoreType.DMA,
            pltpu.SemaphoreType.DMA,
            pltpu.VMEM_SHARED((N_PAD, d), jnp.float32),
        ],
        compiler_params=_SC_PARAMS,
    )
    return fn(y, src2, dst2, zeros)


BLK = 2000


def _mm1_call(x, W1):
    def body(x_ref, w_ref, o_ref):
        o_ref[...] = jnp.dot(x_ref[...], w_ref[...],
                             preferred_element_type=jnp.float32)

    return pl.pallas_call(
        body,
        grid=(N // BLK,),
        in_specs=[pl.BlockSpec((BLK, IN_F), lambda i: (i, 0)),
                  pl.BlockSpec((IN_F, H_F), lambda i: (0, 0))],
        out_specs=pl.BlockSpec((BLK, H_F), lambda i: (i, 0)),
        out_shape=jax.ShapeDtypeStruct((N, H_F), jnp.float32),
    )(x, W1)


def _ew1_call(degp, xw1):
    def body(degp_ref, xw_ref, y_ref, dinv_ref):
        deg = degp_ref[0, :, 0:1] + degp_ref[1, :, 0:1]
        dinv = lax.rsqrt(deg + 1.0)
        y_ref[...] = xw_ref[...] * dinv
        dinv_ref[...] = dinv

    return pl.pallas_call(
        body,
        grid=(N // BLK,),
        in_specs=[pl.BlockSpec((NC, BLK, 16), lambda i: (0, i, 0)),
                  pl.BlockSpec((BLK, H_F), lambda i: (i, 0))],
        out_specs=[pl.BlockSpec((BLK, H_F), lambda i: (i, 0)),
                   pl.BlockSpec((BLK, 1), lambda i: (i, 0))],
        out_shape=[jax.ShapeDtypeStruct((N, H_F), jnp.float32),
                   jax.ShapeDtypeStruct((N, 1), jnp.float32)],
    )(degp, xw1)


def _fused2_call(s1p, y1, dinv, b1, W2):
    def body(s_ref, y_ref, dinv_ref, b_ref, w_ref, o_ref):
        sacc = s_ref[0] + s_ref[1]
        h = jnp.maximum(dinv_ref[...] * (sacc + y_ref[...]) + b_ref[...], 0.0)
        o_ref[...] = jnp.dot(h, w_ref[...],
                             preferred_element_type=jnp.float32) * dinv_ref[...]

    return pl.pallas_call(
        body,
        grid=(N // BLK,),
        in_specs=[pl.BlockSpec((NC, BLK, H_F), lambda i: (0, i, 0)),
                  pl.BlockSpec((BLK, H_F), lambda i: (i, 0)),
                  pl.BlockSpec((BLK, 1), lambda i: (i, 0)),
                  pl.BlockSpec((1, H_F), lambda i: (0, 0)),
                  pl.BlockSpec((H_F, OUT_F), lambda i: (0, 0))],
        out_specs=pl.BlockSpec((BLK, OUT_F), lambda i: (i, 0)),
        out_shape=jax.ShapeDtypeStruct((N, OUT_F), jnp.float32),
    )(s1p, y1, dinv, b1, W2)


def _ew3_call(s2p, y2, dinv, b2):
    def body(s_ref, y_ref, dinv_ref, b_ref, o_ref):
        sacc = s_ref[0] + s_ref[1]
        o_ref[...] = dinv_ref[...] * (sacc + y_ref[...]) + b_ref[...]

    return pl.pallas_call(
        body,
        grid=(N // BLK,),
        in_specs=[pl.BlockSpec((NC, BLK, OUT_F), lambda i: (0, i, 0)),
                  pl.BlockSpec((BLK, OUT_F), lambda i: (i, 0)),
                  pl.BlockSpec((BLK, 1), lambda i: (i, 0)),
                  pl.BlockSpec((1, OUT_F), lambda i: (0, 0))],
        out_specs=pl.BlockSpec((BLK, OUT_F), lambda i: (i, 0)),
        out_shape=jax.ShapeDtypeStruct((N, OUT_F), jnp.float32),
    )(s2p, y2, dinv, b2)


def kernel(x, edge_index, W1, b1, W2, b2):
    src2 = edge_index[0].astype(jnp.int32).reshape(NCHUNKS, CHUNK)
    dst2 = edge_index[1].astype(jnp.int32).reshape(NCHUNKS, CHUNK)

    ones_d = jnp.ones((CHUNK, 16), jnp.float32)
    zeros_d = jnp.zeros((N_PAD, 16), jnp.float32)
    zeros_h = jnp.zeros((N_PAD, H_F), jnp.float32)
    zeros_o = jnp.zeros((N_PAD, OUT_F), jnp.float32)

    degp = _deg_call(dst2, ones_d, zeros_d)
    xw1 = _mm1_call(x, W1)
    y1, dinv = _ew1_call(degp, xw1)
    s1p = _edge_sum_call(y1, src2, dst2, zeros_h, H_F)
    y2 = _fused2_call(s1p, y1, dinv, b1.reshape(1, H_F), W2)
    s2p = _edge_sum_call(y2, src2, dst2, zeros_o, OUT_F)
    out = _ew3_call(s2p, y2, dinv, b2.reshape(1, OUT_F))
    return out


# speedup vs baseline: 2.8051x; 1.0376x over previous
import jax
import jax.numpy as jnp
from jax import lax
from jax.experimental import pallas as pl
from jax.experimental.pallas import tpu as pltpu
from jax.experimental.pallas import tpu_sc as plsc

N = 10000
E = 320000
IN_F = 128
H_F = 128
OUT_F = 64

NC = 2
NS = 16
NW = NC * NS
CHUNK = 128
NCHUNKS = E // CHUNK
NCH_MAX = (NCHUNKS + NW - 1) // NW
N_PAD = 10000
RPT = N_PAD // NS

_MESH = plsc.VectorSubcoreMesh(core_axis_name="c", subcore_axis_name="s")
_SC_PARAMS = pltpu.CompilerParams(use_tc_tiling_on_sc=False,
                                  disable_bounds_checks=True)


def _tile_range(wid):
    lo = wid * NCHUNKS // NW
    hi = (wid + 1) * NCHUNKS // NW
    return lo, hi - lo


def _deg_call(dst2, ones_d, zeros_d):
    def body(dst_hbm, ones_hbm, zeros_hbm, out_hbm, dst_v, ones_v, acc_sh, sem):
        c = lax.axis_index("c")
        s = lax.axis_index("s")
        wid = c * NS + s
        lo, cnt = _tile_range(wid)
        pltpu.sync_copy(zeros_hbm.at[pl.ds(s * RPT, RPT)],
                        acc_sh.at[pl.ds(s * RPT, RPT)])
        pltpu.sync_copy(dst_hbm.at[pl.ds(lo, NCH_MAX)], dst_v)
        pltpu.sync_copy(ones_hbm, ones_v)
        plsc.subcore_barrier()

        @pl.loop(0, NCH_MAX)
        def _(j):
            @pl.when(j < cnt)
            def _():
                pltpu.async_copy(ones_v, acc_sh.at[dst_v.at[j]], sem,
                                 add=True).wait()

        plsc.subcore_barrier()
        pltpu.sync_copy(acc_sh.at[pl.ds(s * RPT, RPT)],
                        out_hbm.at[c].at[pl.ds(s * RPT, RPT)])

    fn = pl.kernel(
        body,
        out_type=jax.ShapeDtypeStruct((NC, N_PAD, 16), jnp.float32),
        mesh=_MESH,
        scratch_types=[
            pltpu.VMEM((NCH_MAX, CHUNK), jnp.int32),
            pltpu.VMEM((CHUNK, 16), jnp.float32),
            pltpu.VMEM_SHARED((N_PAD, 16), jnp.float32),
            pltpu.SemaphoreType.DMA,
        ],
        compiler_params=_SC_PARAMS,
    )
    return fn(dst2, ones_d, zeros_d)


def _edge_sum_call(y, src2, dst2, zeros, d):
    def body(y_hbm, src_hbm, dst_hbm, zeros_hbm, out_hbm,
             src_v, dst_v, rows_a, rows_b, rows_c,
             sem_ia, sem_ib, sem_ic, sem_ga, sem_gb, sem_gc, sem_s, acc_sh):
        c = lax.axis_index("c")
        s = lax.axis_index("s")
        wid = c * NS + s
        lo, cnt = _tile_range(wid)
        pltpu.sync_copy(zeros_hbm.at[pl.ds(s * RPT, RPT)],
                        acc_sh.at[pl.ds(s * RPT, RPT)])
        plsc.subcore_barrier()

        rows = (rows_a, rows_b, rows_c)
        isems = (sem_ia, sem_ib, sem_ic)
        gsems = (sem_ga, sem_gb, sem_gc)

        def i_start(j, slot):
            pltpu.make_async_copy(src_hbm.at[j], src_v.at[slot],
                                  isems[slot]).start()
            pltpu.make_async_copy(dst_hbm.at[j], dst_v.at[slot],
                                  isems[slot]).start()

        def i_wait(j, slot):
            pltpu.make_async_copy(src_hbm.at[j], src_v.at[slot],
                                  isems[slot]).wait()
            pltpu.make_async_copy(dst_hbm.at[j], dst_v.at[slot],
                                  isems[slot]).wait()

        def g_start(slot):
            pltpu.make_async_copy(y_hbm.at[src_v.at[slot]], rows[slot],
                                  gsems[slot]).start()

        def g_wait(slot):
            pltpu.make_async_copy(y_hbm.at[src_v.at[slot]], rows[slot],
                                  gsems[slot]).wait()

        def sc_start(slot):
            return pltpu.async_copy(rows[slot], acc_sh.at[dst_v.at[slot]],
                                    sem_s, add=True)

        i_start(lo, 0)
        i_start(lo + 1, 1)
        i_start(lo + 2, 2)

        @pl.loop(0, (NCH_MAX - 1) // 3)
        def _(jt):
            j = lo + 3 * jt
            i_wait(j, 0)
            g_start(0)
            i_wait(j + 1, 1)
            g_start(1)
            i_wait(j + 2, 2)
            g_start(2)
            g_wait(0)
            sc0 = sc_start(0)
            g_wait(1)
            sc0.wait()

            @pl.when(3 * jt + 3 < cnt)
            def _():
                i_start(j + 3, 0)

            sc1 = sc_start(1)
            g_wait(2)
            sc1.wait()

            @pl.when(3 * jt + 4 < cnt)
            def _():
                i_start(j + 4, 1)

            sc2 = sc_start(2)
            sc2.wait()

            @pl.when(3 * jt + 5 < cnt)
            def _():
                i_start(j + 5, 2)

        @pl.when(cnt > NCH_MAX - 1)
        def _():
            j = lo + NCH_MAX - 1
            i_wait(j, 0)
            g_start(0)
            g_wait(0)
            sc_start(0).wait()

        plsc.subcore_barrier()
        pltpu.sync_copy(acc_sh.at[pl.ds(s * RPT, RPT)],
                        out_hbm.at[c].at[pl.ds(s * RPT, RPT)])

    fn = pl.kernel(
        body,
        out_type=jax.ShapeDtypeStruct((NC, N_PAD, d), jnp.float32),
        mesh=_MESH,
        scratch_types=[
            pltpu.VMEM((3, CHUNK), jnp.int32),
            pltpu.VMEM((3, CHUNK), jnp.int32),
            pltpu.VMEM((CHUNK, d), jnp.float32),
            pltpu.VMEM((CHUNK, d), jnp.float32),
            pltpu.VMEM((CHUNK, d), jnp.float32),
            pltpu.SemaphoreType.DMA,
            pltpu.SemaphoreType.DMA,
            pltpu.SemaphoreType.DMA,
            pltpu.SemaphoreType.DMA,
            pltpu.SemaphoreType.DMA,
            pltpu.SemaphoreType.DMA,
            pltpu.SemaphoreType.DMA,
            pltpu.VMEM_SHARED((N_PAD, d), jnp.float32),
        ],
        compiler_params=_SC_PARAMS,
    )
    return fn(y, src2, dst2, zeros)


BLK = 2000


def _mm1_call(x, W1):
    def body(x_ref, w_ref, o_ref):
        o_ref[...] = jnp.dot(x_ref[...], w_ref[...],
                             preferred_element_type=jnp.float32)

    return pl.pallas_call(
        body,
        grid=(N // BLK,),
        in_specs=[pl.BlockSpec((BLK, IN_F), lambda i: (i, 0)),
                  pl.BlockSpec((IN_F, H_F), lambda i: (0, 0))],
        out_specs=pl.BlockSpec((BLK, H_F), lambda i: (i, 0)),
        out_shape=jax.ShapeDtypeStruct((N, H_F), jnp.float32),
    )(x, W1)


def _ew1_call(degp, xw1):
    def body(degp_ref, xw_ref, y_ref, dinv_ref):
        deg = degp_ref[0, :, 0:1] + degp_ref[1, :, 0:1]
        dinv = lax.rsqrt(deg + 1.0)
        y_ref[...] = xw_ref[...] * dinv
        dinv_ref[...] = dinv

    return pl.pallas_call(
        body,
        grid=(N // BLK,),
        in_specs=[pl.BlockSpec((NC, BLK, 16), lambda i: (0, i, 0)),
                  pl.BlockSpec((BLK, H_F), lambda i: (i, 0))],
        out_specs=[pl.BlockSpec((BLK, H_F), lambda i: (i, 0)),
                   pl.BlockSpec((BLK, 1), lambda i: (i, 0))],
        out_shape=[jax.ShapeDtypeStruct((N, H_F), jnp.float32),
                   jax.ShapeDtypeStruct((N, 1), jnp.float32)],
    )(degp, xw1)


def _fused2_call(s1p, y1, dinv, b1, W2):
    def body(s_ref, y_ref, dinv_ref, b_ref, w_ref, o_ref):
        sacc = s_ref[0] + s_ref[1]
        h = jnp.maximum(dinv_ref[...] * (sacc + y_ref[...]) + b_ref[...], 0.0)
        o_ref[...] = jnp.dot(h, w_ref[...],
                             preferred_element_type=jnp.float32) * dinv_ref[...]

    return pl.pallas_call(
        body,
        grid=(N // BLK,),
        in_specs=[pl.BlockSpec((NC, BLK, H_F), lambda i: (0, i, 0)),
                  pl.BlockSpec((BLK, H_F), lambda i: (i, 0)),
                  pl.BlockSpec((BLK, 1), lambda i: (i, 0)),
                  pl.BlockSpec((1, H_F), lambda i: (0, 0)),
                  pl.BlockSpec((H_F, OUT_F), lambda i: (0, 0))],
        out_specs=pl.BlockSpec((BLK, OUT_F), lambda i: (i, 0)),
        out_shape=jax.ShapeDtypeStruct((N, OUT_F), jnp.float32),
    )(s1p, y1, dinv, b1, W2)


def _ew3_call(s2p, y2, dinv, b2):
    def body(s_ref, y_ref, dinv_ref, b_ref, o_ref):
        sacc = s_ref[0] + s_ref[1]
        o_ref[...] = dinv_ref[...] * (sacc + y_ref[...]) + b_ref[...]

    return pl.pallas_call(
        body,
        grid=(N // BLK,),
        in_specs=[pl.BlockSpec((NC, BLK, OUT_F), lambda i: (0, i, 0)),
                  pl.BlockSpec((BLK, OUT_F), lambda i: (i, 0)),
                  pl.BlockSpec((BLK, 1), lambda i: (i, 0)),
                  pl.BlockSpec((1, OUT_F), lambda i: (0, 0))],
        out_specs=pl.BlockSpec((BLK, OUT_F), lambda i: (i, 0)),
        out_shape=jax.ShapeDtypeStruct((N, OUT_F), jnp.float32),
    )(s2p, y2, dinv, b2)


def kernel(x, edge_index, W1, b1, W2, b2):
    src2 = edge_index[0].astype(jnp.int32).reshape(NCHUNKS, CHUNK)
    dst2 = edge_index[1].astype(jnp.int32).reshape(NCHUNKS, CHUNK)

    ones_d = jnp.ones((CHUNK, 16), jnp.float32)
    zeros_d = jnp.zeros((N_PAD, 16), jnp.float32)
    zeros_h = jnp.zeros((N_PAD, H_F), jnp.float32)
    zeros_o = jnp.zeros((N_PAD, OUT_F), jnp.float32)

    degp = _deg_call(dst2, ones_d, zeros_d)
    xw1 = _mm1_call(x, W1)
    y1, dinv = _ew1_call(degp, xw1)
    s1p = _edge_sum_call(y1, src2, dst2, zeros_h, H_F)
    y2 = _fused2_call(s1p, y1, dinv, b1.reshape(1, H_F), W2)
    s2p = _edge_sum_call(y2, src2, dst2, zeros_o, OUT_F)
    out = _ew3_call(s2p, y2, dinv, b2.reshape(1, OUT_F))
    return out
